# baseline (speedup 1.0000x reference)
import jax
import jax.numpy as jnp
from jax.experimental import pallas as pl

N = 16
N_S = 64
N_A = 8
N_H = 64
N_FC = 64
N_N = 2


def _fused_kernel(ob_ref, mask_ref, fp_ref, states_ref, Wx_ref, bx_ref, Wp_ref,
                  bp_ref, Wm_ref, bm_ref, Wih_ref, Whh_ref, bih_ref, bhh_ref,
                  Wa_ref, ba_ref, Wv_ref, bv_ref, nbr_ref,
                  logits_ref, values_ref, probs_ref, states_out_ref):
    mask = mask_ref[:]
    h = states_ref[:, :N_H] * mask
    c = states_ref[:, N_H:] * mask

    idx = nbr_ref[:]
    iota = jax.lax.broadcasted_iota(jnp.int32, (N, N), 1)
    oh0 = (idx[:, 0:1] == iota).astype(jnp.float32)
    oh1 = (idx[:, 1:2] == iota).astype(jnp.float32)

    ob = ob_ref[:]
    fp = fp_ref[:]

    x_cat = jnp.concatenate(
        [ob, jnp.dot(oh0, ob), jnp.dot(oh1, ob)], axis=1)
    p_i = jnp.concatenate(
        [jnp.dot(oh0, fp), jnp.dot(oh1, fp)], axis=1)
    m_i = jnp.concatenate(
        [jnp.dot(oh0, h), jnp.dot(oh1, h)], axis=1)

    def bmv(W, x):
        return jnp.sum(W * x[:, None, :], axis=2)

    s = jax.nn.relu(bmv(Wx_ref[:], x_cat) + bx_ref[:])
    s = s + jax.nn.relu(bmv(Wp_ref[:], p_i) + bp_ref[:])
    s = s + jax.nn.relu(bmv(Wm_ref[:], m_i) + bm_ref[:])

    gates = bmv(Wih_ref[:], s) + bih_ref[:] + bmv(Whh_ref[:], h) + bhh_ref[:]
    i_g = gates[:, 0 * N_H:1 * N_H]
    f_g = gates[:, 1 * N_H:2 * N_H]
    g_g = gates[:, 2 * N_H:3 * N_H]
    o_g = gates[:, 3 * N_H:4 * N_H]
    c_new = jax.nn.sigmoid(f_g) * c + jax.nn.sigmoid(i_g) * jnp.tanh(g_g)
    h_new = jax.nn.sigmoid(o_g) * jnp.tanh(c_new)

    logits = bmv(Wa_ref[:], h_new) + ba_ref[:]
    values_ref[:] = jnp.sum(Wv_ref[:, 0, :] * h_new, axis=1,
                            keepdims=True) + bv_ref[:]

    logits_ref[:] = logits
    m = jnp.max(logits, axis=1, keepdims=True)
    e = jnp.exp(logits - m)
    probs_ref[:] = e / jnp.sum(e, axis=1, keepdims=True)
    states_out_ref[:] = jnp.concatenate([h_new, c_new], axis=1)


def kernel(ob_N_Do, done_N, fp_N_Dfp, states, Wx, bx, Wp, bp, Wm, bm, Wih,
           Whh, bih, bhh, Wa, ba, Wv, bv, neighbor_idx):
    mask = (1.0 - done_N.astype(jnp.float32))[:, None]
    out_type = (
        jax.ShapeDtypeStruct((N, N_A), jnp.float32),
        jax.ShapeDtypeStruct((N, 1), jnp.float32),
        jax.ShapeDtypeStruct((N, N_A), jnp.float32),
        jax.ShapeDtypeStruct((N, 2 * N_H), jnp.float32),
    )
    logits, values, probs, new_states = pl.pallas_call(
        _fused_kernel,
        out_shape=out_type,
    )(ob_N_Do, mask, fp_N_Dfp, states, Wx, bx, Wp, bp, Wm, bm, Wih, Whh,
      bih, bhh, Wa, ba, Wv, bv, neighbor_idx)
    return (logits, values[:, 0], probs, new_states)
